# Initial kernel scaffold; baseline (speedup 1.0000x reference)
import jax
import jax.numpy as jnp
from jax.experimental import pallas as pl


def kernel(x, edge_index, edge_weight, batch, t_vec, field, W_t, b_t, W_f, W_rel0, W_root0, W_rel1, W_root1, W_m1, b_m1, W_m2, b_m2):
    raise NotImplementedError("write your pallas kernel here")


# speedup vs baseline: 2.9526x; 2.9526x over previous
import jax
import jax.numpy as jnp
from jax import lax
from jax.experimental import pallas as pl
from jax.experimental.pallas import tpu as pltpu
from jax.experimental.pallas import tpu_sc as plsc

_N = 10000
_CH = 128
_G = 16
_EPS = 1e-6

_NC = 2
_NS = 16
_NW = _NC * _NS
_CHK = 128
_NCHT = 80
_EPT = _NCHT * _CHK
_EPAD = _NW * _EPT
_NRCH = 79
_TAIL = _N - (_NRCH - 1) * _CHK
_MPT = 5

_BLK = 2000
_NB = _N // _BLK

_HI = lax.Precision.HIGHEST


def _sig(z):
    return 1.0 / (1.0 + jnp.exp(-z))


def _dot(a, b):
    return lax.dot_general(a, b, (((1,), (0,)), ((), ())), precision=_HI)


def _dotT(a, b):
    return lax.dot_general(a, b, (((0,), (0,)), ((), ())), precision=_HI)


def _onehot(b):
    ids = lax.broadcasted_iota(jnp.int32, (1, _G), 1).astype(jnp.float32)
    return (b == ids).astype(jnp.float32)


def _stats_body(x_ref, b_ref, ms_ref):
    x = x_ref[...]
    oh = _onehot(b_ref[...])
    stats = _dotT(oh, x * x)
    counts = _dotT(oh, jnp.ones_like(b_ref[...]))
    ms_ref[...] = stats / jnp.maximum(counts, 1.0)


def _a2_body(x_ref, b_ref, f_ref, ms_ref, t_ref, wt_ref, bt_ref, wf_ref,
             wm1_ref, bm1_ref, wm2_ref, bm2_ref, wr_ref, wo_ref,
             p_ref, r_ref):
    x = x_ref[...]
    oh = _onehot(b_ref[...])
    inv = lax.rsqrt(_dot(oh, ms_ref[...]) + _EPS)
    h = x * inv
    tv = t_ref[...]
    st = _dot(tv * _sig(tv), wt_ref[...]) + bt_ref[...]
    cond = _dot(oh, st) + f_ref[...] * wf_ref[...]
    gamma = cond[:, :_CH]
    beta = cond[:, _CH:]
    h = h * (1.0 + gamma) + beta
    h = h * _sig(h)
    u = _dot(h, wm1_ref[...]) + bm1_ref[...]
    u = u * _sig(u)
    h2 = h + _dot(u, wm2_ref[...]) + bm2_ref[...]
    p_ref[...] = _dot(h2, wr_ref[...])
    r_ref[...] = _dot(h2, wo_ref[...])


def _b_body(sa_ref, sb_ref, rt_ref, wr_ref, wo_ref, p_ref, r_ref):
    h = sa_ref[...] + sb_ref[...] + rt_ref[...]
    h = h * _sig(h)
    p_ref[...] = _dot(h, wr_ref[...])
    r_ref[...] = _dot(h, wo_ref[...])


def _c_body(sa_ref, sb_ref, rt_ref, x_ref, o_ref):
    o_ref[...] = sa_ref[...] + sb_ref[...] + rt_ref[...] + x_ref[...]


def _sc_scatter(p_hbm, src_hbm, dst_hbm, w_hbm, z_hbm, out_hbm,
                src_v, dst_v, w_v, rows_v, acc_sh, gsem):
    c = lax.axis_index("c")
    s = lax.axis_index("s")
    wid = s * _NC + c
    pltpu.sync_copy(src_hbm.at[wid], src_v)
    pltpu.sync_copy(dst_hbm.at[wid], dst_v)
    pltpu.sync_copy(w_hbm.at[wid], w_v)
    pltpu.sync_copy(z_hbm, rows_v)

    def zbody(m, carry):
        k = s + _NS * m

        @pl.when(k < _NRCH - 1)
        def _full():
            pltpu.sync_copy(rows_v, acc_sh.at[pl.ds(k * _CHK, _CHK)])

        @pl.when(k == _NRCH - 1)
        def _tail():
            pltpu.sync_copy(rows_v.at[pl.ds(0, _TAIL)],
                            acc_sh.at[pl.ds(k * _CHK, _TAIL)])

        return carry

    lax.fori_loop(0, _MPT, zbody, 0)
    plsc.subcore_barrier()

    def body(j, carry):
        pltpu.async_copy(p_hbm.at[src_v.at[j]], rows_v, gsem).wait()

        def mgrp(g, icarry):
            wv = w_v[j, pl.ds(g * 16, 16)]
            for k in range(16):
                wk = wv[k]
                r = g * 16 + k
                for rr in range(8):
                    sl = pl.ds(rr * 16, 16)
                    rows_v[r, sl] = rows_v[r, sl] * wk
            return icarry

        lax.fori_loop(0, _CHK // 16, mgrp, 0)
        pltpu.sync_copy(rows_v, acc_sh.at[dst_v.at[j]], add=True)
        return carry

    lax.fori_loop(0, _NCHT, body, 0)
    plsc.subcore_barrier()

    def drain(m, carry):
        k = s + _NS * m

        @pl.when(k < _NRCH - 1)
        def _full():
            r0 = k * _CHK
            pltpu.sync_copy(acc_sh.at[pl.ds(r0, _CHK)], rows_v)
            pltpu.sync_copy(rows_v, out_hbm.at[c].at[pl.ds(r0, _CHK)])

        @pl.when(k == _NRCH - 1)
        def _tail():
            r0 = k * _CHK
            pltpu.sync_copy(acc_sh.at[pl.ds(r0, _TAIL)], rows_v.at[pl.ds(0, _TAIL)])
            pltpu.sync_copy(rows_v.at[pl.ds(0, _TAIL)],
                            out_hbm.at[c].at[pl.ds(r0, _TAIL)])

        return carry

    lax.fori_loop(0, _MPT, drain, 0)


def _make_sc_call():
    return pl.kernel(
        _sc_scatter,
        out_type=jax.ShapeDtypeStruct((_NC, _N, _CH), jnp.float32),
        mesh=plsc.VectorSubcoreMesh(core_axis_name="c", subcore_axis_name="s"),
        scratch_types=[
            pltpu.VMEM((_NCHT, _CHK), jnp.int32),
            pltpu.VMEM((_NCHT, _CHK), jnp.int32),
            pltpu.VMEM((_NCHT, _CHK), jnp.float32),
            pltpu.VMEM((_CHK, _CH), jnp.float32),
            pltpu.VMEM_SHARED((_N, _CH), jnp.float32),
            pltpu.SemaphoreType.DMA,
        ],
    )


def _row_spec(nb_lanes=_CH):
    return pl.BlockSpec((_BLK, nb_lanes), lambda i: (i, 0))


def _const_spec(shape):
    return pl.BlockSpec(shape, lambda i: (0,) * len(shape))


def kernel(x, edge_index, edge_weight, batch, t_vec, field, W_t, b_t, W_f,
           W_rel0, W_root0, W_rel1, W_root1, W_m1, b_m1, W_m2, b_m2):
    batch_f = batch.astype(jnp.float32).reshape(_N, 1)
    src = edge_index[0]
    dst = edge_index[1]
    w = edge_weight.reshape(-1)
    pad = _EPAD - src.shape[0]
    src_p = jnp.pad(src, (0, pad)).reshape(_NW, _NCHT, _CHK)
    dst_p = jnp.pad(dst, (0, pad)).reshape(_NW, _NCHT, _CHK)
    w_p = jnp.pad(w, (0, pad)).reshape(_NW, _NCHT, _CHK)
    zeros = jnp.zeros((_CHK, _CH), jnp.float32)
    bt2 = b_t.reshape(1, -1)
    bm12 = b_m1.reshape(1, -1)
    bm22 = b_m2.reshape(1, -1)

    ms = pl.pallas_call(
        _stats_body,
        out_shape=jax.ShapeDtypeStruct((_G, _CH), jnp.float32),
    )(x, batch_f)

    f32 = jnp.float32
    p0, r0 = pl.pallas_call(
        _a2_body,
        grid=(_NB,),
        in_specs=[
            _row_spec(), _row_spec(1), _row_spec(1),
            _const_spec((_G, _CH)), _const_spec((_G, _CH)),
            _const_spec((_CH, 2 * _CH)), _const_spec((1, 2 * _CH)),
            _const_spec((1, 2 * _CH)),
            _const_spec((_CH, 2 * _CH)), _const_spec((1, 2 * _CH)),
            _const_spec((2 * _CH, _CH)), _const_spec((1, _CH)),
            _const_spec((_CH, _CH)), _const_spec((_CH, _CH)),
        ],
        out_specs=[_row_spec(), _row_spec()],
        out_shape=[jax.ShapeDtypeStruct((_N, _CH), f32),
                   jax.ShapeDtypeStruct((_N, _CH), f32)],
    )(x, batch_f, field, ms, t_vec, W_t, bt2, W_f, W_m1, bm12, W_m2, bm22,
      W_rel0, W_root0)

    sc_call = _make_sc_call()
    s0 = sc_call(p0, src_p, dst_p, w_p, zeros)

    p1, r1 = pl.pallas_call(
        _b_body,
        grid=(_NB,),
        in_specs=[
            _row_spec(), _row_spec(), _row_spec(),
            _const_spec((_CH, _CH)), _const_spec((_CH, _CH)),
        ],
        out_specs=[_row_spec(), _row_spec()],
        out_shape=[jax.ShapeDtypeStruct((_N, _CH), f32),
                   jax.ShapeDtypeStruct((_N, _CH), f32)],
    )(s0[0], s0[1], r0, W_rel1, W_root1)

    s1 = sc_call(p1, src_p, dst_p, w_p, zeros)

    out = pl.pallas_call(
        _c_body,
        grid=(_NB,),
        in_specs=[_row_spec(), _row_spec(), _row_spec(), _row_spec()],
        out_specs=_row_spec(),
        out_shape=jax.ShapeDtypeStruct((_N, _CH), f32),
    )(s1[0], s1[1], r1, x)
    return out


# speedup vs baseline: 8.0656x; 2.7317x over previous
import jax
import jax.numpy as jnp
from jax import lax
from jax.experimental import pallas as pl
from jax.experimental.pallas import tpu as pltpu
from jax.experimental.pallas import tpu_sc as plsc

_N = 10000
_CH = 128
_G = 16
_EPS = 1e-6

_NC = 2
_NS = 16
_NW = _NC * _NS
_CHK = 80
_NCHT = 125
_EPT = _NCHT * _CHK
_EPAD = _NW * _EPT
_NRCH = _N // _CHK
_MPT = 8

_BLK = 2000
_NB = _N // _BLK

_HI = lax.Precision.HIGHEST


def _sig(z):
    return 1.0 / (1.0 + jnp.exp(-z))


def _dot(a, b):
    return lax.dot_general(a, b, (((1,), (0,)), ((), ())), precision=_HI)


def _dotT(a, b):
    return lax.dot_general(a, b, (((0,), (0,)), ((), ())), precision=_HI)


def _onehot(b):
    ids = lax.broadcasted_iota(jnp.int32, (1, _G), 1).astype(jnp.float32)
    return (b == ids).astype(jnp.float32)


def _stats_body(x_ref, b_ref, ms_ref):
    x = x_ref[...]
    oh = _onehot(b_ref[...])
    stats = _dotT(oh, x * x)
    counts = _dotT(oh, jnp.ones_like(b_ref[...]))
    ms_ref[...] = stats / jnp.maximum(counts, 1.0)


def _a2_body(x_ref, b_ref, f_ref, ms_ref, t_ref, wt_ref, bt_ref, wf_ref,
             wm1_ref, bm1_ref, wm2_ref, bm2_ref, wr_ref, wo_ref,
             p_ref, r_ref):
    x = x_ref[...]
    oh = _onehot(b_ref[...])
    inv = lax.rsqrt(_dot(oh, ms_ref[...]) + _EPS)
    h = x * inv
    tv = t_ref[...]
    st = _dot(tv * _sig(tv), wt_ref[...]) + bt_ref[...]
    cond = _dot(oh, st) + f_ref[...] * wf_ref[...]
    gamma = cond[:, :_CH]
    beta = cond[:, _CH:]
    h = h * (1.0 + gamma) + beta
    h = h * _sig(h)
    u = _dot(h, wm1_ref[...]) + bm1_ref[...]
    u = u * _sig(u)
    h2 = h + _dot(u, wm2_ref[...]) + bm2_ref[...]
    p_ref[...] = _dot(h2, wr_ref[...])
    r_ref[...] = _dot(h2, wo_ref[...])


def _b_body(sa_ref, sb_ref, rt_ref, wr_ref, wo_ref, p_ref, r_ref):
    h = sa_ref[...] + sb_ref[...] + rt_ref[...]
    h = h * _sig(h)
    p_ref[...] = _dot(h, wr_ref[...])
    r_ref[...] = _dot(h, wo_ref[...])


def _c_body(sa_ref, sb_ref, rt_ref, x_ref, o_ref):
    o_ref[...] = sa_ref[...] + sb_ref[...] + rt_ref[...] + x_ref[...]


def _sc_scatter(p_hbm, src_hbm, dst_hbm, w_hbm, z_hbm, out_hbm,
                src_v, dst_v, w_v, dstc_v, rows_v, rows2_v, acc_sh, gsem, gsem2):
    c = lax.axis_index("c")
    s = lax.axis_index("s")
    wid = s * _NC + c
    e0 = wid * _EPT
    pltpu.sync_copy(src_hbm.at[pl.ds(e0, _EPT)], src_v)
    pltpu.sync_copy(dst_hbm.at[pl.ds(e0, _EPT)], dst_v)
    pltpu.sync_copy(w_hbm.at[pl.ds(e0, _EPT)], w_v)
    pltpu.sync_copy(z_hbm, rows_v)

    def zbody(m, carry):
        k = s + _NS * m

        @pl.when(k < _NRCH)
        def _full():
            pltpu.sync_copy(rows_v, acc_sh.at[pl.ds(k * _CHK, _CHK)])

        return carry

    lax.fori_loop(0, _MPT, zbody, 0)
    plsc.subcore_barrier()

    bufs = (rows_v, rows2_v)
    sems = (gsem, gsem2)

    def _gather(j, buf, sem):
        return pltpu.make_async_copy(
            p_hbm.at[src_v.at[pl.ds(j * _CHK, _CHK)]], buf, sem)

    _gather(0, bufs[0], sems[0]).start()
    _gather(1, bufs[1], sems[1]).start()

    def _proc(j, buf):
        def mgrp(g, icarry):
            base = j * _CHK + g * 16
            wv = w_v[pl.ds(base, 16)]
            dstc_v[pl.ds(g * 16, 16)] = dst_v[pl.ds(base, 16)]
            for k in range(16):
                wk = wv[k]
                r = g * 16 + k
                for rr in range(8):
                    sl = pl.ds(rr * 16, 16)
                    buf[r, sl] = buf[r, sl] * wk
            return icarry

        lax.fori_loop(0, _CHK // 16, mgrp, 0)
        pltpu.sync_copy(buf, acc_sh.at[dstc_v], add=True)

    def body(jj, carry):
        for b in range(2):
            j = 2 * jj + b
            buf = bufs[b]
            sem = sems[b]
            _gather(j, buf, sem).wait()
            _proc(j, buf)

            @pl.when(j + 2 < _NCHT)
            def _pref():
                _gather(j + 2, buf, sem).start()

        return carry

    lax.fori_loop(0, _NCHT // 2, body, 0)
    if _NCHT % 2:
        jlast = _NCHT - 1
        _gather(jlast, bufs[0], sems[0]).wait()
        _proc(jlast, bufs[0])
    plsc.subcore_barrier()

    def drain(m, carry):
        k = s + _NS * m

        @pl.when(k < _NRCH)
        def _full():
            r0 = k * _CHK
            pltpu.sync_copy(acc_sh.at[pl.ds(r0, _CHK)], rows_v)
            pltpu.sync_copy(rows_v, out_hbm.at[c].at[pl.ds(r0, _CHK)])

        return carry

    lax.fori_loop(0, _MPT, drain, 0)


def _make_sc_call():
    return pl.kernel(
        _sc_scatter,
        out_type=jax.ShapeDtypeStruct((_NC, _N, _CH), jnp.float32),
        mesh=plsc.VectorSubcoreMesh(core_axis_name="c", subcore_axis_name="s"),
        scratch_types=[
            pltpu.VMEM((_EPT,), jnp.int32),
            pltpu.VMEM((_EPT,), jnp.int32),
            pltpu.VMEM((_EPT,), jnp.float32),
            pltpu.VMEM((_CHK,), jnp.int32),
            pltpu.VMEM((_CHK, _CH), jnp.float32),
            pltpu.VMEM((_CHK, _CH), jnp.float32),
            pltpu.VMEM_SHARED((_N, _CH), jnp.float32),
            pltpu.SemaphoreType.DMA,
            pltpu.SemaphoreType.DMA,
        ],
    )


def _row_spec(nb_lanes=_CH):
    return pl.BlockSpec((_BLK, nb_lanes), lambda i: (i, 0))


def _const_spec(shape):
    return pl.BlockSpec(shape, lambda i: (0,) * len(shape))


def kernel(x, edge_index, edge_weight, batch, t_vec, field, W_t, b_t, W_f,
           W_rel0, W_root0, W_rel1, W_root1, W_m1, b_m1, W_m2, b_m2):
    batch_f = batch.astype(jnp.float32).reshape(_N, 1)
    src_p = edge_index[0]
    dst_p = edge_index[1]
    w_p = edge_weight.reshape(-1)
    zeros = jnp.zeros((_CHK, _CH), jnp.float32)
    bt2 = b_t.reshape(1, -1)
    bm12 = b_m1.reshape(1, -1)
    bm22 = b_m2.reshape(1, -1)

    ms = pl.pallas_call(
        _stats_body,
        out_shape=jax.ShapeDtypeStruct((_G, _CH), jnp.float32),
    )(x, batch_f)

    f32 = jnp.float32
    p0, r0 = pl.pallas_call(
        _a2_body,
        grid=(_NB,),
        in_specs=[
            _row_spec(), _row_spec(1), _row_spec(1),
            _const_spec((_G, _CH)), _const_spec((_G, _CH)),
            _const_spec((_CH, 2 * _CH)), _const_spec((1, 2 * _CH)),
            _const_spec((1, 2 * _CH)),
            _const_spec((_CH, 2 * _CH)), _const_spec((1, 2 * _CH)),
            _const_spec((2 * _CH, _CH)), _const_spec((1, _CH)),
            _const_spec((_CH, _CH)), _const_spec((_CH, _CH)),
        ],
        out_specs=[_row_spec(), _row_spec()],
        out_shape=[jax.ShapeDtypeStruct((_N, _CH), f32),
                   jax.ShapeDtypeStruct((_N, _CH), f32)],
    )(x, batch_f, field, ms, t_vec, W_t, bt2, W_f, W_m1, bm12, W_m2, bm22,
      W_rel0, W_root0)

    sc_call = _make_sc_call()
    s0 = sc_call(p0, src_p, dst_p, w_p, zeros)

    p1, r1 = pl.pallas_call(
        _b_body,
        grid=(_NB,),
        in_specs=[
            _row_spec(), _row_spec(), _row_spec(),
            _const_spec((_CH, _CH)), _const_spec((_CH, _CH)),
        ],
        out_specs=[_row_spec(), _row_spec()],
        out_shape=[jax.ShapeDtypeStruct((_N, _CH), f32),
                   jax.ShapeDtypeStruct((_N, _CH), f32)],
    )(s0[0], s0[1], r0, W_rel1, W_root1)

    s1 = sc_call(p1, src_p, dst_p, w_p, zeros)

    out = pl.pallas_call(
        _c_body,
        grid=(_NB,),
        in_specs=[_row_spec(), _row_spec(), _row_spec(), _row_spec()],
        out_specs=_row_spec(),
        out_shape=jax.ShapeDtypeStruct((_N, _CH), f32),
    )(s1[0], s1[1], r1, x)
    return out
